# baseline (speedup 1.0000x reference)
import functools

import jax
import jax.numpy as jnp
from jax import lax
from jax.experimental import pallas as pl
from jax.experimental.pallas import tpu as pltpu
from jax.experimental.pallas import tpu_sc as plsc

_VOCAB = 100000
_EMBED = 32
_B = 4096
_L = 200
_N = _B * _L

_NC = 2
_NS = 16
_NW = _NC * _NS
_PER_W = _N // _NW
_LBLK = 4
_ROWS = _LBLK * 128
_NBLK = _L // _LBLK
_NPAIR = _NBLK // 2


def _emb_body(table_hbm, idx_hbm, out_hbm,
              idx_raw, idx_t, gbuf0, gbuf1, tbuf0, tbuf1, sem0, sem1):
    wid = lax.axis_index("s") * _NC + lax.axis_index("c")
    iota = lax.iota(jnp.int32, 16)
    bufs = ((gbuf0, tbuf0, sem0), (gbuf1, tbuf1, sem1))

    pltpu.sync_copy(idx_hbm.at[pl.ds(wid * _PER_W, _PER_W)], idx_raw)

    @plsc.parallel_loop(0, _L, 1, unroll=2)
    def idx_t_body(l):
        for j in range(8):
            v = plsc.load_gather(idx_raw, [l + (j * 16 + iota) * _L])
            idx_t[pl.ds(l * 128 + j * 16, 16)] = v

    def fire(k, p):
        gbuf, _, sem = bufs[p]
        pltpu.async_copy(table_hbm.at[idx_t.at[pl.ds(k * _ROWS, _ROWS)]],
                         gbuf, sem)

    def drain_transpose_store(k, p):
        gbuf, tbuf, sem = bufs[p]
        pltpu.make_async_copy(table_hbm.at[idx_t.at[pl.ds(k * _ROWS, _ROWS)]],
                              gbuf, sem).wait()

        @plsc.parallel_loop(0, 8, 1, unroll=4)
        def tr_body(j):
            for lq in range(_LBLK):
                for e in range(_EMBED):
                    rows = lq * 128 + j * 16 + iota
                    v = plsc.load_gather(gbuf, [rows, jnp.full((16,), e, jnp.int32)])
                    tbuf[lq, e // 8, 0,
                         pl.ds((e % 8) * 128 + j * 16, 16)] = v
        pltpu.sync_copy(
            tbuf,
            out_hbm.at[pl.ds(k * _LBLK, _LBLK), slice(None),
                       pl.ds(wid, 1), slice(None)])

    fire(0, 0)

    def pair(q, _):
        for p in range(2):
            k = 2 * q + p
            if p == 0:
                fire(k + 1, 1)
            else:
                @pl.when(q < _NPAIR - 1)
                def _():
                    fire(k + 1, 0)
            drain_transpose_store(k, p)
        return 0

    lax.fori_loop(0, _NPAIR, pair, 0)


_emb = functools.partial(
    pl.kernel,
    mesh=plsc.VectorSubcoreMesh(core_axis_name="c", subcore_axis_name="s"),
    out_type=jax.ShapeDtypeStruct((_L, 4, _NW, 1024), jnp.float32),
    scratch_types=[
        pltpu.VMEM((_PER_W,), jnp.int32),
        pltpu.VMEM((_PER_W,), jnp.int32),
        pltpu.VMEM((_ROWS, _EMBED), jnp.float32),
        pltpu.VMEM((_ROWS, _EMBED), jnp.float32),
        pltpu.VMEM((_LBLK, 4, 1, 1024), jnp.float32),
        pltpu.VMEM((_LBLK, 4, 1, 1024), jnp.float32),
        pltpu.SemaphoreType.DMA,
        pltpu.SemaphoreType.DMA,
    ],
    compiler_params=pltpu.CompilerParams(use_tc_tiling_on_sc=False,
                                         needs_layout_passes=False),
)(_emb_body)


def kernel(indices, table):
    flat_idx = indices.reshape(_N)
    out5 = _emb(table, flat_idx).reshape(_L, 4, _NW, 8, 128)
    return out5.transpose(2, 4, 0, 1, 3).reshape(_B, _L, _EMBED)


# speedup vs baseline: 2.8202x; 2.8202x over previous
import functools

import jax
import jax.numpy as jnp
from jax import lax
from jax.experimental import pallas as pl
from jax.experimental.pallas import tpu as pltpu
from jax.experimental.pallas import tpu_sc as plsc

_VOCAB = 100000
_EMBED = 32
_B = 4096
_L = 200
_N = _B * _L

_NC = 2
_NS = 16
_NW = _NC * _NS
_PER_W = _N // _NW
_LBLK = 4
_ROWS = _LBLK * 128
_NBLK = _L // _LBLK
_NPAIR = _NBLK // 2


def _emb_body(table_hbm, idx_hbm, out_hbm,
              idx_raw, idx_t, gbuf0, gbuf1, tbuf0, tbuf1, sem0, sem1):
    wid = lax.axis_index("s") * _NC + lax.axis_index("c")
    iota = lax.iota(jnp.int32, 16)
    bufs = ((gbuf0, tbuf0, sem0), (gbuf1, tbuf1, sem1))
    zeros16 = jnp.zeros((16,), jnp.int32)

    pltpu.sync_copy(idx_hbm.at[pl.ds(wid * _PER_W, _PER_W)], idx_raw)

    @plsc.parallel_loop(0, _L, 1, unroll=2)
    def idx_t_body(l):
        for j in range(8):
            v = plsc.load_gather(idx_raw, [l + (j * 16 + iota) * _L])
            idx_t[pl.ds(l * 128 + j * 16, 16)] = v

    def fire(k, p):
        gbuf, _, sem = bufs[p]
        pltpu.async_copy(table_hbm.at[idx_t.at[pl.ds(k * _ROWS, _ROWS)]],
                         gbuf, sem)

    def drain_transpose_store(k, p):
        gbuf, tbuf, sem = bufs[p]
        pltpu.make_async_copy(table_hbm.at[idx_t.at[pl.ds(k * _ROWS, _ROWS)]],
                              gbuf, sem).wait()

        @plsc.parallel_loop(0, 8, 1, unroll=2)
        def tr_body(j):
            for lq in range(_LBLK):
                rows = lq * 128 + j * 16 + iota
                for e0 in range(_EMBED):
                    evec = (e0 + iota) % _EMBED
                    v = plsc.load_gather(gbuf, [rows, evec])
                    qv = lq * 4 + evec // 8
                    cv = (evec % 8) * 128 + j * 16 + iota
                    plsc.store_scatter(tbuf, [qv, zeros16, cv], v)

        pltpu.sync_copy(
            tbuf,
            out_hbm.at[pl.ds(k * 16, 16), pl.ds(wid, 1), slice(None)])

    fire(0, 0)

    def pair(q, _):
        for p in range(2):
            k = 2 * q + p
            if p == 0:
                fire(k + 1, 1)
            else:
                @pl.when(q < _NPAIR - 1)
                def _():
                    fire(k + 1, 0)
            drain_transpose_store(k, p)
        return 0

    lax.fori_loop(0, _NPAIR, pair, 0)


_emb = functools.partial(
    pl.kernel,
    mesh=plsc.VectorSubcoreMesh(core_axis_name="c", subcore_axis_name="s"),
    out_type=jax.ShapeDtypeStruct((_L * 4, _NW, 1024), jnp.float32),
    scratch_types=[
        pltpu.VMEM((_PER_W,), jnp.int32),
        pltpu.VMEM((_PER_W,), jnp.int32),
        pltpu.VMEM((_ROWS, _EMBED), jnp.float32),
        pltpu.VMEM((_ROWS, _EMBED), jnp.float32),
        pltpu.VMEM((16, 1, 1024), jnp.float32),
        pltpu.VMEM((16, 1, 1024), jnp.float32),
        pltpu.SemaphoreType.DMA,
        pltpu.SemaphoreType.DMA,
    ],
    compiler_params=pltpu.CompilerParams(use_tc_tiling_on_sc=False,
                                         needs_layout_passes=False),
)(_emb_body)


def kernel(indices, table):
    flat_idx = indices.reshape(_N)
    out5 = _emb(table, flat_idx).reshape(_L, 4, _NW, 8, 128)
    return out5.transpose(2, 4, 0, 1, 3).reshape(_B, _L, _EMBED)
